# baseline (speedup 1.0000x reference)
import functools

import jax
import jax.numpy as jnp
from jax import lax
from jax.experimental import pallas as pl
from jax.experimental.pallas import tpu as pltpu
from jax.experimental.pallas import tpu_sc as plsc

_B = 256
_D = 4 * 64 * 64
_LANES = 16


def _allsc_body(a_hbm, b_hbm, t_hbm, x_hbm, n_hbm, o_hbm,
                idx_v, rows_a, rows_b, xv, nv, ov, sg, sx, sn, so):
    info = plsc.get_sparse_core_info()
    nc = info.num_cores
    wid = lax.axis_index("s") * nc + lax.axis_index("c")
    nw = nc * info.num_subcores
    bw = _B // nw
    base = wid * bw

    pltpu.sync_copy(t_hbm.at[pl.ds(base, bw)], idx_v)
    cp_a = pltpu.async_copy(a_hbm.at[idx_v], rows_a.at[pl.ds(0, bw)], sg)
    cp_b = pltpu.async_copy(b_hbm.at[idx_v], rows_b.at[pl.ds(0, bw)], sg)
    cp_a.wait()
    cp_b.wait()
    va = rows_a[...]
    vb = rows_b[...]

    def start_in(r):
        s = r % 2
        pltpu.make_async_copy(x_hbm.at[base + r], xv.at[s], sx.at[s]).start()
        pltpu.make_async_copy(n_hbm.at[base + r], nv.at[s], sn.at[s]).start()

    start_in(0)
    start_in(1)
    for r in range(bw):
        s = r % 2
        pltpu.make_async_copy(x_hbm.at[base + r], xv.at[s], sx.at[s]).wait()
        pltpu.make_async_copy(n_hbm.at[base + r], nv.at[s], sn.at[s]).wait()
        if r >= 2:
            pltpu.make_async_copy(
                ov.at[s], o_hbm.at[base + r - 2], so.at[s]
            ).wait()
        a_s = va[r]
        b_s = vb[r]

        def chunk(j, _):
            sl = pl.ds(j * _LANES, _LANES)
            ov.at[s][sl] = a_s * xv.at[s][sl] + b_s * nv.at[s][sl]
            return _

        lax.fori_loop(0, _D // _LANES, chunk, 0, unroll=8)
        pltpu.make_async_copy(ov.at[s], o_hbm.at[base + r], so.at[s]).start()
        if r + 2 < bw:
            start_in(r + 2)
    for r in range(bw - 2, bw):
        s = r % 2
        pltpu.make_async_copy(ov.at[s], o_hbm.at[base + r], so.at[s]).wait()


def _allsc(a_tbl, b_tbl, t, x2, n2):
    mesh = plsc.VectorSubcoreMesh(core_axis_name="c", subcore_axis_name="s")
    info = plsc.get_sparse_core_info()
    nw = info.num_cores * info.num_subcores
    bw = _B // nw
    f = functools.partial(
        pl.kernel,
        mesh=mesh,
        out_type=jax.ShapeDtypeStruct((_B, _D), jnp.float32),
        scratch_types=[
            pltpu.VMEM((bw,), jnp.int32),
            pltpu.VMEM((_LANES,), jnp.float32),
            pltpu.VMEM((_LANES,), jnp.float32),
            pltpu.VMEM((2, _D), jnp.float32),
            pltpu.VMEM((2, _D), jnp.float32),
            pltpu.VMEM((2, _D), jnp.float32),
            pltpu.SemaphoreType.DMA,
            pltpu.SemaphoreType.DMA((2,)),
            pltpu.SemaphoreType.DMA((2,)),
            pltpu.SemaphoreType.DMA((2,)),
        ],
    )(_allsc_body)
    return f(a_tbl, b_tbl, t, x2, n2)


def kernel(x_start, t, noise, sqrt_alphas_cumprod, sqrt_one_minus_alphas_cumprod):
    x2 = x_start.reshape(_B, _D)
    n2 = noise.reshape(_B, _D)
    out = _allsc(
        sqrt_alphas_cumprod.astype(jnp.float32),
        sqrt_one_minus_alphas_cumprod.astype(jnp.float32),
        t.astype(jnp.int32),
        x2,
        n2,
    )
    return out.reshape(x_start.shape)


# speedup vs baseline: 1.4557x; 1.4557x over previous
import functools

import jax
import jax.numpy as jnp
from jax import lax
from jax.experimental import pallas as pl
from jax.experimental.pallas import tpu as pltpu
from jax.experimental.pallas import tpu_sc as plsc

_B = 256
_D = 4 * 64 * 64
_LANES = 16


def _allsc_body(a_hbm, b_hbm, t_hbm, x_hbm, n_hbm, o_hbm,
                idx_v, rows_a, rows_b, xv, nv, ov, sg, sx, sn, so):
    info = plsc.get_sparse_core_info()
    nc = info.num_cores
    wid = lax.axis_index("s") * nc + lax.axis_index("c")
    nw = nc * info.num_subcores
    bw = _B // nw
    base = wid * bw

    pltpu.sync_copy(t_hbm.at[pl.ds(base, bw)], idx_v)
    cp_a = pltpu.async_copy(a_hbm.at[idx_v], rows_a.at[pl.ds(0, bw)], sg)
    cp_b = pltpu.async_copy(b_hbm.at[idx_v], rows_b.at[pl.ds(0, bw)], sg)
    cp_a.wait()
    cp_b.wait()
    va = rows_a[...]
    vb = rows_b[...]

    def start_in(r):
        s = r % 2
        pltpu.make_async_copy(x_hbm.at[base + r], xv.at[s], sx.at[s]).start()
        pltpu.make_async_copy(n_hbm.at[base + r], nv.at[s], sn.at[s]).start()

    start_in(0)
    start_in(1)
    for r in range(bw):
        s = r % 2
        pltpu.make_async_copy(x_hbm.at[base + r], xv.at[s], sx.at[s]).wait()
        pltpu.make_async_copy(n_hbm.at[base + r], nv.at[s], sn.at[s]).wait()
        if r >= 2:
            pltpu.make_async_copy(
                ov.at[s], o_hbm.at[base + r - 2], so.at[s]
            ).wait()
        a_s = va[r]
        b_s = vb[r]
        xv_s, nv_s, ov_s = xv.at[s], nv.at[s], ov.at[s]

        @plsc.parallel_loop(0, _D, step=_LANES, unroll=8)
        def _(off):
            sl = pl.ds(off, _LANES)
            ov_s[sl] = a_s * xv_s[sl] + b_s * nv_s[sl]
        pltpu.make_async_copy(ov.at[s], o_hbm.at[base + r], so.at[s]).start()
        if r + 2 < bw:
            start_in(r + 2)
    for r in range(bw - 2, bw):
        s = r % 2
        pltpu.make_async_copy(ov.at[s], o_hbm.at[base + r], so.at[s]).wait()


def _allsc(a_tbl, b_tbl, t, x2, n2):
    mesh = plsc.VectorSubcoreMesh(core_axis_name="c", subcore_axis_name="s")
    info = plsc.get_sparse_core_info()
    nw = info.num_cores * info.num_subcores
    bw = _B // nw
    f = functools.partial(
        pl.kernel,
        mesh=mesh,
        out_type=jax.ShapeDtypeStruct((_B, _D), jnp.float32),
        scratch_types=[
            pltpu.VMEM((bw,), jnp.int32),
            pltpu.VMEM((_LANES,), jnp.float32),
            pltpu.VMEM((_LANES,), jnp.float32),
            pltpu.VMEM((2, _D), jnp.float32),
            pltpu.VMEM((2, _D), jnp.float32),
            pltpu.VMEM((2, _D), jnp.float32),
            pltpu.SemaphoreType.DMA,
            pltpu.SemaphoreType.DMA((2,)),
            pltpu.SemaphoreType.DMA((2,)),
            pltpu.SemaphoreType.DMA((2,)),
        ],
    )(_allsc_body)
    return f(a_tbl, b_tbl, t, x2, n2)


def kernel(x_start, t, noise, sqrt_alphas_cumprod, sqrt_one_minus_alphas_cumprod):
    x2 = x_start.reshape(_B, _D)
    n2 = noise.reshape(_B, _D)
    out = _allsc(
        sqrt_alphas_cumprod.astype(jnp.float32),
        sqrt_one_minus_alphas_cumprod.astype(jnp.float32),
        t.astype(jnp.int32),
        x2,
        n2,
    )
    return out.reshape(x_start.shape)


# speedup vs baseline: 1.4561x; 1.0003x over previous
import functools

import jax
import jax.numpy as jnp
from jax import lax
from jax.experimental import pallas as pl
from jax.experimental.pallas import tpu as pltpu
from jax.experimental.pallas import tpu_sc as plsc

_B = 256
_D = 4 * 64 * 64
_LANES = 16


def _allsc_body(a_hbm, b_hbm, t_hbm, x_hbm, n_hbm, o_hbm,
                idx_v, rows_a, rows_b, xv, nv, ov, sg, sx, sn, so):
    info = plsc.get_sparse_core_info()
    nc = info.num_cores
    wid = lax.axis_index("s") * nc + lax.axis_index("c")
    nw = nc * info.num_subcores
    bw = _B // nw
    base = wid * bw

    pltpu.sync_copy(t_hbm.at[pl.ds(base, bw)], idx_v)
    cp_a = pltpu.async_copy(a_hbm.at[idx_v], rows_a.at[pl.ds(0, bw)], sg)
    cp_b = pltpu.async_copy(b_hbm.at[idx_v], rows_b.at[pl.ds(0, bw)], sg)
    cp_a.wait()
    cp_b.wait()
    va = rows_a[...]
    vb = rows_b[...]

    def start_in(r):
        s = r % 2
        pltpu.make_async_copy(x_hbm.at[base + r], xv.at[s], sx.at[s]).start()
        pltpu.make_async_copy(n_hbm.at[base + r], nv.at[s], sn.at[s]).start()

    start_in(0)
    start_in(1)
    for r in range(bw):
        s = r % 2
        pltpu.make_async_copy(x_hbm.at[base + r], xv.at[s], sx.at[s]).wait()
        pltpu.make_async_copy(n_hbm.at[base + r], nv.at[s], sn.at[s]).wait()
        if r >= 2:
            pltpu.make_async_copy(
                ov.at[s], o_hbm.at[base + r - 2], so.at[s]
            ).wait()
        a_s = va[r]
        b_s = vb[r]
        xv_s, nv_s, ov_s = xv.at[s], nv.at[s], ov.at[s]

        @plsc.parallel_loop(0, _D, step=_LANES, unroll=16)
        def _(off):
            sl = pl.ds(off, _LANES)
            ov_s[sl] = a_s * xv_s[sl] + b_s * nv_s[sl]
        pltpu.make_async_copy(ov.at[s], o_hbm.at[base + r], so.at[s]).start()
        if r + 2 < bw:
            start_in(r + 2)
    for r in range(bw - 2, bw):
        s = r % 2
        pltpu.make_async_copy(ov.at[s], o_hbm.at[base + r], so.at[s]).wait()


def _allsc(a_tbl, b_tbl, t, x2, n2):
    mesh = plsc.VectorSubcoreMesh(core_axis_name="c", subcore_axis_name="s")
    info = plsc.get_sparse_core_info()
    nw = info.num_cores * info.num_subcores
    bw = _B // nw
    f = functools.partial(
        pl.kernel,
        mesh=mesh,
        out_type=jax.ShapeDtypeStruct((_B, _D), jnp.float32),
        scratch_types=[
            pltpu.VMEM((bw,), jnp.int32),
            pltpu.VMEM((_LANES,), jnp.float32),
            pltpu.VMEM((_LANES,), jnp.float32),
            pltpu.VMEM((2, _D), jnp.float32),
            pltpu.VMEM((2, _D), jnp.float32),
            pltpu.VMEM((2, _D), jnp.float32),
            pltpu.SemaphoreType.DMA,
            pltpu.SemaphoreType.DMA((2,)),
            pltpu.SemaphoreType.DMA((2,)),
            pltpu.SemaphoreType.DMA((2,)),
        ],
    )(_allsc_body)
    return f(a_tbl, b_tbl, t, x2, n2)


def kernel(x_start, t, noise, sqrt_alphas_cumprod, sqrt_one_minus_alphas_cumprod):
    x2 = x_start.reshape(_B, _D)
    n2 = noise.reshape(_B, _D)
    out = _allsc(
        sqrt_alphas_cumprod.astype(jnp.float32),
        sqrt_one_minus_alphas_cumprod.astype(jnp.float32),
        t.astype(jnp.int32),
        x2,
        n2,
    )
    return out.reshape(x_start.shape)


# speedup vs baseline: 1.5753x; 1.0818x over previous
import functools

import jax
import jax.numpy as jnp
from jax import lax
from jax.experimental import pallas as pl
from jax.experimental.pallas import tpu as pltpu
from jax.experimental.pallas import tpu_sc as plsc

_B = 256
_D = 4 * 64 * 64
_NCH = 32
_RW = _B // _NCH
_NBUF = 4


def _sc_gather_body(a_hbm, b_hbm, t_hbm, a_out, b_out, idx_v, rows_a, rows_b, sem):
    info = plsc.get_sparse_core_info()
    nc = info.num_cores
    wid = lax.axis_index("s") * nc + lax.axis_index("c")
    nw = nc * info.num_subcores
    bw = _B // nw
    base = wid * bw
    pltpu.sync_copy(t_hbm.at[pl.ds(base, bw)], idx_v)
    cp_a = pltpu.async_copy(a_hbm.at[idx_v], rows_a, sem)
    cp_b = pltpu.async_copy(b_hbm.at[idx_v], rows_b, sem)
    cp_a.wait()
    cp_b.wait()
    pltpu.sync_copy(rows_a, a_out.at[pl.ds(base, bw)])
    pltpu.sync_copy(rows_b, b_out.at[pl.ds(base, bw)])


def _sc_gather(a_tbl, b_tbl, t):
    info = plsc.get_sparse_core_info()
    nw = info.num_cores * info.num_subcores
    bw = _B // nw
    mesh = plsc.VectorSubcoreMesh(core_axis_name="c", subcore_axis_name="s")
    f = functools.partial(
        pl.kernel,
        mesh=mesh,
        out_type=(
            jax.ShapeDtypeStruct((_B,), jnp.float32),
            jax.ShapeDtypeStruct((_B,), jnp.float32),
        ),
        scratch_types=[
            pltpu.VMEM((bw,), jnp.int32),
            pltpu.VMEM((bw,), jnp.float32),
            pltpu.VMEM((bw,), jnp.float32),
            pltpu.SemaphoreType.DMA,
        ],
    )(_sc_gather_body)
    return f(a_tbl, b_tbl, t)


def _fma_body(a_ref, b_ref, x_hbm, n_hbm, o_hbm, xv, nv, ov, sx, sn, so):
    def start_in(c):
        s = c % _NBUF
        pltpu.make_async_copy(x_hbm.at[pl.ds(c * _RW, _RW)], xv.at[s], sx.at[s]).start()
        pltpu.make_async_copy(n_hbm.at[pl.ds(c * _RW, _RW)], nv.at[s], sn.at[s]).start()

    for c in range(_NBUF):
        start_in(c)
    for c in range(_NCH):
        s = c % _NBUF
        pltpu.make_async_copy(x_hbm.at[pl.ds(c * _RW, _RW)], xv.at[s], sx.at[s]).wait()
        pltpu.make_async_copy(n_hbm.at[pl.ds(c * _RW, _RW)], nv.at[s], sn.at[s]).wait()
        if c >= _NBUF:
            pltpu.make_async_copy(
                ov.at[s], o_hbm.at[pl.ds((c - _NBUF) * _RW, _RW)], so.at[s]
            ).wait()
        a = a_ref[pl.ds(c * _RW, _RW), :]
        b = b_ref[pl.ds(c * _RW, _RW), :]
        ov[s] = a * xv[s] + b * nv[s]
        pltpu.make_async_copy(ov.at[s], o_hbm.at[pl.ds(c * _RW, _RW)], so.at[s]).start()
        if c + _NBUF < _NCH:
            start_in(c + _NBUF)
    for c in range(_NCH - _NBUF, _NCH):
        s = c % _NBUF
        pltpu.make_async_copy(ov.at[s], o_hbm.at[pl.ds(c * _RW, _RW)], so.at[s]).wait()


def _fma(a_g, b_g, x2, n2):
    return pl.pallas_call(
        _fma_body,
        in_specs=[
            pl.BlockSpec(memory_space=pltpu.VMEM),
            pl.BlockSpec(memory_space=pltpu.VMEM),
            pl.BlockSpec(memory_space=pl.ANY),
            pl.BlockSpec(memory_space=pl.ANY),
        ],
        out_specs=pl.BlockSpec(memory_space=pl.ANY),
        out_shape=jax.ShapeDtypeStruct((_B, _D), jnp.float32),
        scratch_shapes=[
            pltpu.VMEM((_NBUF, _RW, _D), jnp.float32),
            pltpu.VMEM((_NBUF, _RW, _D), jnp.float32),
            pltpu.VMEM((_NBUF, _RW, _D), jnp.float32),
            pltpu.SemaphoreType.DMA((_NBUF,)),
            pltpu.SemaphoreType.DMA((_NBUF,)),
            pltpu.SemaphoreType.DMA((_NBUF,)),
        ],
    )(a_g, b_g, x2, n2)


def kernel(x_start, t, noise, sqrt_alphas_cumprod, sqrt_one_minus_alphas_cumprod):
    a_g, b_g = _sc_gather(
        sqrt_alphas_cumprod.astype(jnp.float32),
        sqrt_one_minus_alphas_cumprod.astype(jnp.float32),
        t.astype(jnp.int32),
    )
    x2 = x_start.reshape(_B, _D)
    n2 = noise.reshape(_B, _D)
    out = _fma(a_g.reshape(_B, 1), b_g.reshape(_B, 1), x2, n2)
    return out.reshape(x_start.shape)
